# baseline (speedup 1.0000x reference)
import functools

import jax
import jax.numpy as jnp
from jax import lax
from jax.experimental import pallas as pl
from jax.experimental.pallas import tpu as pltpu
from jax.experimental.pallas import tpu_sc as plsc

_ROWS = 128
_COLS = 32768
_K = 64
_SUB = 8
_LANE = 128


@functools.cache
def _make_gather():
    info = plsc.get_sparse_core_info()
    nc, ns = info.num_cores, info.num_subcores
    n_blocks = _ROWS // _SUB

    mesh = plsc.VectorSubcoreMesh(core_axis_name="c", subcore_axis_name="s")

    @functools.partial(
        pl.kernel,
        mesh=mesh,
        out_type=jax.ShapeDtypeStruct((_ROWS, _K), jnp.float32),
        scratch_types=[
            pltpu.VMEM((_K,), jnp.int32),
            pltpu.VMEM((_K,), jnp.int32),
            pltpu.VMEM((_K * _SUB, _LANE), jnp.float32),
            pltpu.VMEM((_SUB, _K), jnp.float32),
            pltpu.SemaphoreType.DMA,
        ],
        compiler_params=pltpu.CompilerParams(needs_layout_passes=False),
    )
    def gather_kernel(
        x_hbm, mask_hbm, out_hbm, mask_v, lanes_v, blocks_v, vals_v, sem
    ):
        wid = lax.axis_index("s") * nc + lax.axis_index("c")

        @pl.when(wid < n_blocks)
        def _():
            pltpu.sync_copy(mask_hbm, mask_v)
            r0 = pl.multiple_of(wid * _SUB, _SUB)
            iota = lax.iota(jnp.int32, 16)
            row = lax.bitwise_and(iota, 7)
            half = lax.shift_right_logical(iota, 3)

            for c in range(_K // 16):
                lanes_v[pl.ds(16 * c, 16)] = lax.bitwise_and(
                    mask_v[pl.ds(16 * c, 16)], 127
                )

            def mask_scalar(j):
                pos = jnp.broadcast_to(j, (16,))
                return plsc.load_gather(mask_v, [pos])[0]

            def issue(j, carry):
                m = mask_scalar(j)
                mt = pl.multiple_of(
                    lax.shift_left(lax.shift_right_logical(m, 7), 7), _LANE
                )
                dst = pl.multiple_of(j * _SUB, _SUB)
                pltpu.async_copy(
                    x_hbm.at[pl.ds(r0, _SUB), pl.ds(mt, _LANE)],
                    blocks_v.at[pl.ds(dst, _SUB)],
                    sem,
                )
                return carry

            lax.fori_loop(0, _K, issue, 0, unroll=8)

            for q in range(4):
                pltpu.make_async_copy(
                    x_hbm.at[:, pl.ds(0, _LANE)],
                    blocks_v.at[pl.ds(q * _ROWS, _ROWS)],
                    sem,
                ).wait()

            def extract(p, carry):
                j0 = 2 * p
                col = half + j0
                lane = plsc.load_gather(lanes_v, [col])
                vec = plsc.load_gather(blocks_v, [col * _SUB + row, lane])
                plsc.store_scatter(vals_v, [row, col], vec)
                return carry

            lax.fori_loop(0, _K // 2, extract, 0, unroll=4)
            pltpu.sync_copy(vals_v, out_hbm.at[pl.ds(r0, _SUB), :])

    return gather_kernel


def kernel(x, mask):
    return _make_gather()(x, mask)


# speedup vs baseline: 1.0117x; 1.0117x over previous
import functools

import jax
import jax.numpy as jnp
from jax import lax
from jax.experimental import pallas as pl
from jax.experimental.pallas import tpu as pltpu
from jax.experimental.pallas import tpu_sc as plsc

_ROWS = 128
_COLS = 32768
_K = 64
_SUB = 8
_LANE = 128


@functools.cache
def _make_gather():
    info = plsc.get_sparse_core_info()
    nc, ns = info.num_cores, info.num_subcores
    n_blocks = _ROWS // _SUB

    mesh = plsc.VectorSubcoreMesh(core_axis_name="c", subcore_axis_name="s")

    @functools.partial(
        pl.kernel,
        mesh=mesh,
        out_type=jax.ShapeDtypeStruct((_ROWS, _K), jnp.float32),
        scratch_types=[
            pltpu.VMEM((_K,), jnp.int32),
            pltpu.VMEM((_K,), jnp.int32),
            pltpu.VMEM((_K * _SUB, _LANE), jnp.float32),
            pltpu.VMEM((_SUB, _K), jnp.float32),
            pltpu.SemaphoreType.DMA,
        ],
        compiler_params=pltpu.CompilerParams(needs_layout_passes=False),
    )
    def gather_kernel(
        x_hbm, mask_hbm, out_hbm, mask_v, lanes_v, blocks_v, vals_v, sem
    ):
        wid = lax.axis_index("s") * nc + lax.axis_index("c")

        @pl.when(wid < n_blocks)
        def _():
            pltpu.sync_copy(mask_hbm, mask_v)
            r0 = pl.multiple_of(wid * _SUB, _SUB)
            iota = lax.iota(jnp.int32, 16)
            row = lax.bitwise_and(iota, 7)
            half = lax.shift_right_logical(iota, 3)

            for c in range(_K // 16):
                lanes_v[pl.ds(16 * c, 16)] = lax.bitwise_and(
                    mask_v[pl.ds(16 * c, 16)], 127
                )

            def mask_scalar(j):
                pos = jnp.broadcast_to(j, (16,))
                return plsc.load_gather(mask_v, [pos])[0]

            def issue(j, carry):
                m = mask_scalar(j)
                mt = pl.multiple_of(
                    lax.shift_left(lax.shift_right_logical(m, 7), 7), _LANE
                )
                dst = pl.multiple_of(j * _SUB, _SUB)
                pltpu.async_copy(
                    x_hbm.at[pl.ds(r0, _SUB), pl.ds(mt, _LANE)],
                    blocks_v.at[pl.ds(dst, _SUB)],
                    sem,
                )
                return carry

            with jax.named_scope("issue_dmas"):
                lax.fori_loop(0, _K, issue, 0)

            with jax.named_scope("drain"):
                for q in range(4):
                    pltpu.make_async_copy(
                        x_hbm.at[:, pl.ds(0, _LANE)],
                        blocks_v.at[pl.ds(q * _ROWS, _ROWS)],
                        sem,
                    ).wait()

            def extract(p, carry):
                j0 = 2 * p
                col = half + j0
                lane = plsc.load_gather(lanes_v, [col])
                vec = plsc.load_gather(blocks_v, [col * _SUB + row, lane])
                plsc.store_scatter(vals_v, [row, col], vec)
                return carry

            with jax.named_scope("extract"):
                lax.fori_loop(0, _K // 2, extract, 0)
            with jax.named_scope("writeback"):
                pltpu.sync_copy(vals_v, out_hbm.at[pl.ds(r0, _SUB), :])

    return gather_kernel


def kernel(x, mask):
    return _make_gather()(x, mask)


# speedup vs baseline: 1.0447x; 1.0326x over previous
import functools

import jax
import jax.numpy as jnp
from jax import lax
from jax.experimental import pallas as pl
from jax.experimental.pallas import tpu as pltpu
from jax.experimental.pallas import tpu_sc as plsc

_ROWS = 128
_COLS = 32768
_K = 64
_SUB = 8
_LANE = 128
_HALF = _K // 2


@functools.cache
def _make_gather():
    info = plsc.get_sparse_core_info()
    nc, ns = info.num_cores, info.num_subcores

    mesh = plsc.VectorSubcoreMesh(core_axis_name="c", subcore_axis_name="s")

    @functools.partial(
        pl.kernel,
        mesh=mesh,
        out_type=jax.ShapeDtypeStruct((_ROWS, _K), jnp.float32),
        scratch_types=[
            pltpu.VMEM((_K,), jnp.int32),
            pltpu.VMEM((_K,), jnp.int32),
            pltpu.VMEM((_HALF * _SUB, _LANE), jnp.float32),
            pltpu.VMEM((_HALF * _SUB,), jnp.float32),
            pltpu.VMEM((_HALF * _SUB,), jnp.float32),
            pltpu.VMEM((_SUB, _K), jnp.float32),
            pltpu.VMEM_SHARED((8, _HALF * _SUB), jnp.float32),
            pltpu.SemaphoreType.DMA,
        ],
        compiler_params=pltpu.CompilerParams(needs_layout_passes=False),
    )
    def gather_kernel(
        x_hbm,
        mask_hbm,
        out_hbm,
        mask_v,
        lanes_v,
        blocks_v,
        frag_v,
        frag2_v,
        vals_v,
        shared,
        sem,
    ):
        s = lax.axis_index("s")
        c = lax.axis_index("c")
        h = lax.bitwise_and(s, 1)
        pid = lax.shift_right_logical(s, 1)
        r0 = pl.multiple_of((c * (ns // 2) + pid) * _SUB, _SUB)
        jb = h * _HALF

        pltpu.sync_copy(mask_hbm, mask_v)
        iota = lax.iota(jnp.int32, 16)
        row = lax.bitwise_and(iota, 7)
        half16 = lax.shift_right_logical(iota, 3)

        for q in range(_K // 16):
            lanes_v[pl.ds(16 * q, 16)] = lax.bitwise_and(
                mask_v[pl.ds(16 * q, 16)], 127
            )

        def mask_scalar(j):
            pos = jnp.broadcast_to(j, (16,))
            return plsc.load_gather(mask_v, [pos])[0]

        def issue(jj, carry):
            m = mask_scalar(jb + jj)
            mt = pl.multiple_of(
                lax.shift_left(lax.shift_right_logical(m, 7), 7), _LANE
            )
            dst = pl.multiple_of(jj * _SUB, _SUB)
            pltpu.async_copy(
                x_hbm.at[pl.ds(r0, _SUB), pl.ds(mt, _LANE)],
                blocks_v.at[pl.ds(dst, _SUB)],
                sem,
            )
            return carry

        lax.fori_loop(0, _HALF, issue, 0)

        for q in range(2):
            pltpu.make_async_copy(
                x_hbm.at[:, pl.ds(0, _LANE)],
                blocks_v.at[pl.ds(q * _ROWS, _ROWS)],
                sem,
            ).wait()

        def extract(p, carry):
            cl = 2 * p + half16
            lane = plsc.load_gather(lanes_v, [jb + cl])
            vec = plsc.load_gather(blocks_v, [cl * _SUB + row, lane])
            plsc.store_scatter(frag_v, [row * _HALF + cl], vec)
            return carry

        lax.fori_loop(0, _HALF // 2, extract, 0)

        @pl.when(h == 1)
        def _():
            pltpu.sync_copy(frag_v, shared.at[pid])

        plsc.subcore_barrier()

        @pl.when(h == 0)
        def _():
            pltpu.sync_copy(shared.at[pid], frag2_v)

            def merge(k, carry):
                src = pl.multiple_of(16 * k, 16)
                p16 = iota + 16 * k
                rr = lax.shift_right_logical(p16, 5)
                cc = lax.bitwise_and(p16, _HALF - 1)
                plsc.store_scatter(
                    vals_v, [rr, cc], frag_v[pl.ds(src, 16)]
                )
                plsc.store_scatter(
                    vals_v, [rr, cc + _HALF], frag2_v[pl.ds(src, 16)]
                )
                return carry

            lax.fori_loop(0, _HALF * _SUB // 16, merge, 0)
            pltpu.sync_copy(vals_v, out_hbm.at[pl.ds(r0, _SUB), :])

    return gather_kernel


def kernel(x, mask):
    return _make_gather()(x, mask)


# speedup vs baseline: 1.1238x; 1.0757x over previous
import functools

import jax
import jax.numpy as jnp
from jax import lax
from jax.experimental import pallas as pl
from jax.experimental.pallas import tpu as pltpu
from jax.experimental.pallas import tpu_sc as plsc

_ROWS = 128
_COLS = 32768
_K = 64
_SUB = 8
_LANE = 128
_CPT = 2


@functools.cache
def _make_gather():
    info = plsc.get_sparse_core_info()
    nc, ns = info.num_cores, info.num_subcores

    mesh = plsc.VectorSubcoreMesh(core_axis_name="c", subcore_axis_name="s")

    @functools.partial(
        pl.kernel,
        mesh=mesh,
        out_type=jax.ShapeDtypeStruct((_K, _ROWS), jnp.float32),
        scratch_types=[
            pltpu.VMEM((_K,), jnp.int32),
            pltpu.VMEM((_K,), jnp.int32),
            pltpu.VMEM((_CPT * _ROWS, _LANE), jnp.float32),
            pltpu.VMEM((_CPT, _ROWS), jnp.float32),
            pltpu.VMEM((_SUB, _ROWS), jnp.float32),
            pltpu.VMEM_SHARED((ns, _CPT, _ROWS), jnp.float32),
            pltpu.SemaphoreType.DMA,
        ],
        compiler_params=pltpu.CompilerParams(needs_layout_passes=False),
    )
    def gather_kernel(
        x_hbm,
        mask_hbm,
        out_hbm,
        mask_v,
        lanes_v,
        blocks_v,
        frag_v,
        vals_v,
        shared,
        sem,
    ):
        s = lax.axis_index("s")
        c = lax.axis_index("c")
        b = c * 4 + lax.shift_right_logical(s, 2)
        q = lax.bitwise_and(s, 3)
        j0 = b * _SUB + q * _CPT

        pltpu.sync_copy(mask_hbm, mask_v)
        iota = lax.iota(jnp.int32, 16)

        for u in range(_K // 16):
            lanes_v[pl.ds(16 * u, 16)] = lax.bitwise_and(
                mask_v[pl.ds(16 * u, 16)], 127
            )

        def mask_scalar(j):
            pos = jnp.broadcast_to(j, (16,))
            return plsc.load_gather(mask_v, [pos])[0]

        for t in range(_CPT):
            m = mask_scalar(j0 + t)
            mt = pl.multiple_of(
                lax.shift_left(lax.shift_right_logical(m, 7), 7), _LANE
            )
            pltpu.async_copy(
                x_hbm.at[:, pl.ds(mt, _LANE)],
                blocks_v.at[pl.ds(t * _ROWS, _ROWS)],
                sem,
            )

        for t in range(_CPT):
            pltpu.make_async_copy(
                x_hbm.at[:, pl.ds(0, _LANE)],
                blocks_v.at[pl.ds(t * _ROWS, _ROWS)],
                sem,
            ).wait()

        for t in range(_CPT):
            lane = plsc.load_gather(lanes_v, [jnp.broadcast_to(j0 + t, (16,))])

            def pick(k, carry):
                base = pl.multiple_of(16 * k, 16)
                vec = plsc.load_gather(
                    blocks_v, [t * _ROWS + base + iota, lane]
                )
                frag_v[t, pl.ds(base, 16)] = vec
                return carry

            lax.fori_loop(0, _ROWS // 16, pick, 0)

        pltpu.sync_copy(frag_v, shared.at[s])
        plsc.subcore_barrier()

        @pl.when(q == 0)
        def _():
            for g in range(_SUB // _CPT):
                pltpu.sync_copy(
                    shared.at[s + g], vals_v.at[pl.ds(g * _CPT, _CPT)]
                )
            pltpu.sync_copy(
                vals_v, out_hbm.at[pl.ds(pl.multiple_of(b * _SUB, _SUB), _SUB), :]
            )

    return gather_kernel


def kernel(x, mask):
    return _make_gather()(x, mask).T
